# Initial kernel scaffold; baseline (speedup 1.0000x reference)
import jax
import jax.numpy as jnp
from jax.experimental import pallas as pl


def kernel(x, edge_index, batch, W1, b1, W2, b2):
    raise NotImplementedError("write your pallas kernel here")


# speedup vs baseline: 14.7446x; 14.7446x over previous
import functools

import jax
import jax.numpy as jnp
from jax import lax
from jax.experimental import pallas as pl
from jax.experimental.pallas import tpu as pltpu
from jax.experimental.pallas import tpu_sc as plsc

NC, NS = 2, 16
NW = NC * NS
CHUNK = 128
RBLK = 1000


def _wid(c, s):
    return c * NS + s


def _deg_body(n_pad, rows_per_tile, cpt, dst_hbm, degp_hbm, dst_v, ones_v,
              init_v, deg_sp, sem):
    c = lax.axis_index("c")
    s = lax.axis_index("s")
    w = _wid(c, s)
    val = jnp.where(c == 0, 1.0, 0.0).astype(jnp.float32)
    for j in range(rows_per_tile // 16):
        init_v[pl.ds(j * 16, 16)] = jnp.full((16,), 1.0, jnp.float32) * val
    for j in range(CHUNK // 16):
        ones_v[pl.ds(j * 16, 16)] = jnp.full((16,), 1.0, jnp.float32)
    pltpu.sync_copy(init_v, deg_sp.at[pl.ds(s * rows_per_tile, rows_per_tile)])
    pltpu.sync_copy(dst_hbm.at[w], dst_v)
    plsc.subcore_barrier()

    def body(j, carry):
        pltpu.sync_copy(ones_v, deg_sp.at[dst_v.at[j]], add=True)
        return carry

    lax.fori_loop(0, cpt, body, 0)
    plsc.subcore_barrier()
    pltpu.sync_copy(deg_sp.at[pl.ds(s * rows_per_tile, rows_per_tile)],
                    degp_hbm.at[c, pl.ds(s * rows_per_tile, rows_per_tile)])


def _y_body(x_ref, w1_ref, d0_ref, d1_ref, y_ref, dinv_ref):
    deg = d0_ref[...] + d1_ref[...]
    dinv = lax.rsqrt(deg)
    xw = jnp.dot(x_ref[...], w1_ref[...], preferred_element_type=jnp.float32)
    y_ref[...] = xw * dinv
    dinv_ref[...] = dinv


def _edge_body(n_pad, rows_per_tile, cpt, src_hbm, dst_hbm, y_hbm, zeros_hbm,
               acc_hbm, src_v, dst_v, rows_v, acc_sp, sem):
    c = lax.axis_index("c")
    s = lax.axis_index("s")
    w = _wid(c, s)
    pltpu.sync_copy(zeros_hbm, acc_sp.at[pl.ds(s * rows_per_tile, rows_per_tile)])
    pltpu.sync_copy(src_hbm.at[w], src_v)
    pltpu.sync_copy(dst_hbm.at[w], dst_v)
    plsc.subcore_barrier()

    def body(j, carry):
        pltpu.async_copy(y_hbm.at[src_v.at[j]], rows_v, sem).wait()
        pltpu.sync_copy(rows_v, acc_sp.at[dst_v.at[j]], add=True)
        return carry

    lax.fori_loop(0, cpt, body, 0)
    plsc.subcore_barrier()
    pltpu.sync_copy(acc_sp.at[pl.ds(s * rows_per_tile, rows_per_tile)],
                    acc_hbm.at[c, pl.ds(s * rows_per_tile, rows_per_tile)])


def _final_body(g_segs, a0_ref, a1_ref, y_ref, dinv_ref, batch_ref, b1_ref,
                w2_ref, b2_ref, out_ref, pooled_ref):
    i = pl.program_id(0)

    @pl.when(i == 0)
    def _init():
        pooled_ref[...] = jnp.zeros_like(pooled_ref)

    h = jnp.maximum(
        dinv_ref[...] * (a0_ref[...] + a1_ref[...] + y_ref[...]) + b1_ref[...],
        0.0)
    bids = batch_ref[...]

    def seg(g, carry):
        contrib = jnp.max(jnp.where(bids == g, h, 0.0), axis=0, keepdims=True)
        cur = pooled_ref[pl.ds(g, 1), :]
        pooled_ref[pl.ds(g, 1), :] = jnp.maximum(cur, contrib)
        return carry

    lax.fori_loop(0, g_segs, seg, 0)

    @pl.when(i == pl.num_programs(0) - 1)
    def _head():
        p = pooled_ref[...]
        out_ref[...] = jnp.maximum(
            jnp.dot(p, w2_ref[...], preferred_element_type=jnp.float32)
            + b2_ref[...], 0.0)


def kernel(x, edge_index, batch, W1, b1, W2, b2):
    n, f = x.shape
    h_dim = W1.shape[1]
    o_dim = W2.shape[1]
    g_segs = 64
    e = edge_index.shape[1]

    cpt = -(-e // (NW * CHUNK))
    e_pad = NW * cpt * CHUNK
    n_pad = -(-n // (NS * 16)) * (NS * 16)
    rows_per_tile = n_pad // NS
    dummy = n_pad - 1

    src = edge_index[0]
    dst = edge_index[1]
    pad = e_pad - e
    src_p = jnp.concatenate(
        [src, jnp.zeros((pad,), jnp.int32)]).reshape(NW, cpt, CHUNK)
    dst_p = jnp.concatenate(
        [dst, jnp.full((pad,), dummy, jnp.int32)]).reshape(NW, cpt, CHUNK)

    mesh = plsc.VectorSubcoreMesh(core_axis_name="c", subcore_axis_name="s")

    deg_kernel = pl.kernel(
        functools.partial(_deg_body, n_pad, rows_per_tile, cpt),
        out_type=jax.ShapeDtypeStruct((NC, n_pad), jnp.float32),
        mesh=mesh,
        scratch_types=[
            pltpu.VMEM((cpt, CHUNK), jnp.int32),
            pltpu.VMEM((CHUNK,), jnp.float32),
            pltpu.VMEM((rows_per_tile,), jnp.float32),
            pltpu.VMEM_SHARED((n_pad,), jnp.float32),
            pltpu.SemaphoreType.DMA,
        ],
    )
    degp = deg_kernel(dst_p)
    d0 = degp[0, :n].reshape(n, 1)
    d1 = degp[1, :n].reshape(n, 1)

    y, dinv = pl.pallas_call(
        _y_body,
        grid=(n // RBLK,),
        in_specs=[
            pl.BlockSpec((RBLK, f), lambda i: (i, 0)),
            pl.BlockSpec((f, h_dim), lambda i: (0, 0)),
            pl.BlockSpec((RBLK, 1), lambda i: (i, 0)),
            pl.BlockSpec((RBLK, 1), lambda i: (i, 0)),
        ],
        out_specs=[
            pl.BlockSpec((RBLK, h_dim), lambda i: (i, 0)),
            pl.BlockSpec((RBLK, 1), lambda i: (i, 0)),
        ],
        out_shape=[
            jax.ShapeDtypeStruct((n, h_dim), jnp.float32),
            jax.ShapeDtypeStruct((n, 1), jnp.float32),
        ],
    )(x, W1, d0, d1)

    zeros_rows = jnp.zeros((rows_per_tile, h_dim), jnp.float32)
    edge_kernel = pl.kernel(
        functools.partial(_edge_body, n_pad, rows_per_tile, cpt),
        out_type=jax.ShapeDtypeStruct((NC, n_pad, h_dim), jnp.float32),
        mesh=mesh,
        scratch_types=[
            pltpu.VMEM((cpt, CHUNK), jnp.int32),
            pltpu.VMEM((cpt, CHUNK), jnp.int32),
            pltpu.VMEM((CHUNK, h_dim), jnp.float32),
            pltpu.VMEM_SHARED((n_pad, h_dim), jnp.float32),
            pltpu.SemaphoreType.DMA,
        ],
    )
    acc = edge_kernel(src_p, dst_p, y, zeros_rows)
    a0 = acc[0, :n, :]
    a1 = acc[1, :n, :]

    out = pl.pallas_call(
        functools.partial(_final_body, g_segs),
        grid=(n // RBLK,),
        in_specs=[
            pl.BlockSpec((RBLK, h_dim), lambda i: (i, 0)),
            pl.BlockSpec((RBLK, h_dim), lambda i: (i, 0)),
            pl.BlockSpec((RBLK, h_dim), lambda i: (i, 0)),
            pl.BlockSpec((RBLK, 1), lambda i: (i, 0)),
            pl.BlockSpec((RBLK, 1), lambda i: (i, 0)),
            pl.BlockSpec((1, h_dim), lambda i: (0, 0)),
            pl.BlockSpec((h_dim, o_dim), lambda i: (0, 0)),
            pl.BlockSpec((1, o_dim), lambda i: (0, 0)),
        ],
        out_specs=pl.BlockSpec((g_segs, o_dim), lambda i: (0, 0)),
        out_shape=jax.ShapeDtypeStruct((g_segs, o_dim), jnp.float32),
        scratch_shapes=[pltpu.VMEM((g_segs, h_dim), jnp.float32)],
    )(a0, a1, y, dinv, batch.reshape(n, 1), b1.reshape(1, h_dim), W2,
      b2.reshape(1, o_dim))
    return out


# speedup vs baseline: 18.9629x; 1.2861x over previous
import functools

import jax
import jax.numpy as jnp
from jax import lax
from jax.experimental import pallas as pl
from jax.experimental.pallas import tpu as pltpu
from jax.experimental.pallas import tpu_sc as plsc

NC, NS = 2, 16
NW = NC * NS
CHUNK = 128
RBLK = 1000


def _wid(c, s):
    return c * NS + s


def _deg_body(n_pad, rows_per_tile, cpt, dst_hbm, degp_hbm, dst_v, ones_v,
              init_v, deg_sp, sem):
    c = lax.axis_index("c")
    s = lax.axis_index("s")
    w = _wid(c, s)
    val = jnp.where(c == 0, 1.0, 0.0).astype(jnp.float32)
    for j in range(rows_per_tile // 16):
        init_v[pl.ds(j * 16, 16)] = jnp.full((16,), 1.0, jnp.float32) * val
    for j in range(CHUNK // 16):
        ones_v[pl.ds(j * 16, 16)] = jnp.full((16,), 1.0, jnp.float32)
    pltpu.sync_copy(init_v, deg_sp.at[pl.ds(s * rows_per_tile, rows_per_tile)])
    pltpu.sync_copy(dst_hbm.at[w], dst_v)
    plsc.subcore_barrier()

    def body(j, carry):
        pltpu.sync_copy(ones_v, deg_sp.at[dst_v.at[j]], add=True)
        return carry

    lax.fori_loop(0, cpt, body, 0)
    plsc.subcore_barrier()
    pltpu.sync_copy(deg_sp.at[pl.ds(s * rows_per_tile, rows_per_tile)],
                    degp_hbm.at[c, pl.ds(s * rows_per_tile, rows_per_tile)])


def _y_body(x_ref, w1_ref, d0_ref, d1_ref, y_ref, dinv_ref):
    deg = d0_ref[...] + d1_ref[...]
    dinv = lax.rsqrt(deg)
    xw = jnp.dot(x_ref[...], w1_ref[...], preferred_element_type=jnp.float32)
    y_ref[...] = xw * dinv
    dinv_ref[...] = dinv


def _edge_body(n_pad, rows_per_tile, cpt, src_hbm, dst_hbm, y_hbm, zeros_hbm,
               acc_hbm, src_v, dst_v, rows_v, acc_sp, sem):
    c = lax.axis_index("c")
    s = lax.axis_index("s")
    w = _wid(c, s)
    pltpu.sync_copy(zeros_hbm, acc_sp.at[pl.ds(s * rows_per_tile, rows_per_tile)])
    pltpu.sync_copy(src_hbm.at[w], src_v)
    pltpu.sync_copy(dst_hbm.at[w], dst_v)
    plsc.subcore_barrier()

    def body(j, carry):
        pltpu.async_copy(y_hbm.at[src_v.at[j]], rows_v, sem).wait()
        pltpu.sync_copy(rows_v, acc_sp.at[dst_v.at[j]], add=True)
        return carry

    lax.fori_loop(0, cpt, body, 0)
    plsc.subcore_barrier()
    pltpu.sync_copy(acc_sp.at[pl.ds(s * rows_per_tile, rows_per_tile)],
                    acc_hbm.at[c, pl.ds(s * rows_per_tile, rows_per_tile)])


def _final_body(g_segs, a0_ref, a1_ref, y_ref, dinv_ref, batch_ref, b1_ref,
                w2_ref, b2_ref, out_ref, pooled_ref):
    i = pl.program_id(0)

    @pl.when(i == 0)
    def _init():
        pooled_ref[...] = jnp.zeros_like(pooled_ref)

    h = jnp.maximum(
        dinv_ref[...] * (a0_ref[...] + a1_ref[...] + y_ref[...]) + b1_ref[...],
        0.0)
    bids = batch_ref[...]

    def seg(g, carry):
        contrib = jnp.max(jnp.where(bids == g, h, 0.0), axis=0, keepdims=True)
        cur = pooled_ref[pl.ds(g, 1), :]
        pooled_ref[pl.ds(g, 1), :] = jnp.maximum(cur, contrib)
        return carry

    g_lo = batch_ref[0, 0]
    g_hi = batch_ref[batch_ref.shape[0] - 1, 0]
    lax.fori_loop(g_lo, g_hi + 1, seg, 0)

    @pl.when(i == pl.num_programs(0) - 1)
    def _head():
        p = pooled_ref[...]
        out_ref[...] = jnp.maximum(
            jnp.dot(p, w2_ref[...], preferred_element_type=jnp.float32)
            + b2_ref[...], 0.0)


def kernel(x, edge_index, batch, W1, b1, W2, b2):
    n, f = x.shape
    h_dim = W1.shape[1]
    o_dim = W2.shape[1]
    g_segs = 64
    e = edge_index.shape[1]

    cpt = -(-e // (NW * CHUNK))
    e_pad = NW * cpt * CHUNK
    n_pad = -(-n // (NS * 16)) * (NS * 16)
    rows_per_tile = n_pad // NS
    dummy = n_pad - 1

    src = edge_index[0]
    dst = edge_index[1]
    pad = e_pad - e
    src_p = jnp.concatenate(
        [src, jnp.zeros((pad,), jnp.int32)]).reshape(NW, cpt, CHUNK)
    dst_p = jnp.concatenate(
        [dst, jnp.full((pad,), dummy, jnp.int32)]).reshape(NW, cpt, CHUNK)

    mesh = plsc.VectorSubcoreMesh(core_axis_name="c", subcore_axis_name="s")

    deg_kernel = pl.kernel(
        functools.partial(_deg_body, n_pad, rows_per_tile, cpt),
        out_type=jax.ShapeDtypeStruct((NC, n_pad), jnp.float32),
        mesh=mesh,
        scratch_types=[
            pltpu.VMEM((cpt, CHUNK), jnp.int32),
            pltpu.VMEM((CHUNK,), jnp.float32),
            pltpu.VMEM((rows_per_tile,), jnp.float32),
            pltpu.VMEM_SHARED((n_pad,), jnp.float32),
            pltpu.SemaphoreType.DMA,
        ],
    )
    degp = deg_kernel(dst_p)
    d0 = degp[0, :n].reshape(n, 1)
    d1 = degp[1, :n].reshape(n, 1)

    y, dinv = pl.pallas_call(
        _y_body,
        grid=(n // RBLK,),
        in_specs=[
            pl.BlockSpec((RBLK, f), lambda i: (i, 0)),
            pl.BlockSpec((f, h_dim), lambda i: (0, 0)),
            pl.BlockSpec((RBLK, 1), lambda i: (i, 0)),
            pl.BlockSpec((RBLK, 1), lambda i: (i, 0)),
        ],
        out_specs=[
            pl.BlockSpec((RBLK, h_dim), lambda i: (i, 0)),
            pl.BlockSpec((RBLK, 1), lambda i: (i, 0)),
        ],
        out_shape=[
            jax.ShapeDtypeStruct((n, h_dim), jnp.float32),
            jax.ShapeDtypeStruct((n, 1), jnp.float32),
        ],
    )(x, W1, d0, d1)

    zeros_rows = jnp.zeros((rows_per_tile, h_dim), jnp.float32)
    edge_kernel = pl.kernel(
        functools.partial(_edge_body, n_pad, rows_per_tile, cpt),
        out_type=jax.ShapeDtypeStruct((NC, n_pad, h_dim), jnp.float32),
        mesh=mesh,
        scratch_types=[
            pltpu.VMEM((cpt, CHUNK), jnp.int32),
            pltpu.VMEM((cpt, CHUNK), jnp.int32),
            pltpu.VMEM((CHUNK, h_dim), jnp.float32),
            pltpu.VMEM_SHARED((n_pad, h_dim), jnp.float32),
            pltpu.SemaphoreType.DMA,
        ],
    )
    acc = edge_kernel(src_p, dst_p, y, zeros_rows)
    a0 = acc[0, :n, :]
    a1 = acc[1, :n, :]

    out = pl.pallas_call(
        functools.partial(_final_body, g_segs),
        grid=(n // RBLK,),
        in_specs=[
            pl.BlockSpec((RBLK, h_dim), lambda i: (i, 0)),
            pl.BlockSpec((RBLK, h_dim), lambda i: (i, 0)),
            pl.BlockSpec((RBLK, h_dim), lambda i: (i, 0)),
            pl.BlockSpec((RBLK, 1), lambda i: (i, 0)),
            pl.BlockSpec((RBLK, 1), lambda i: (i, 0)),
            pl.BlockSpec((1, h_dim), lambda i: (0, 0)),
            pl.BlockSpec((h_dim, o_dim), lambda i: (0, 0)),
            pl.BlockSpec((1, o_dim), lambda i: (0, 0)),
        ],
        out_specs=pl.BlockSpec((g_segs, o_dim), lambda i: (0, 0)),
        out_shape=jax.ShapeDtypeStruct((g_segs, o_dim), jnp.float32),
        scratch_shapes=[pltpu.VMEM((g_segs, h_dim), jnp.float32)],
    )(a0, a1, y, dinv, batch.reshape(n, 1), b1.reshape(1, h_dim), W2,
      b2.reshape(1, o_dim))
    return out


# speedup vs baseline: 22.3662x; 1.1795x over previous
import functools

import jax
import jax.numpy as jnp
from jax import lax
from jax.experimental import pallas as pl
from jax.experimental.pallas import tpu as pltpu
from jax.experimental.pallas import tpu_sc as plsc

NC, NS = 2, 16
NW = NC * NS
CHUNK = 80
RBLK = 1000


def _wid(c, s):
    return c * NS + s


def _deg_body(n_pad, rows_per_tile, cpt, dst_hbm, degp_hbm, dst_v, ones_v,
              init_v, deg_sp, sem):
    c = lax.axis_index("c")
    s = lax.axis_index("s")
    w = _wid(c, s)
    val = jnp.where(c == 0, 1.0, 0.0).astype(jnp.float32)
    for j in range(rows_per_tile // 16):
        init_v[pl.ds(j * 16, 16)] = jnp.full((16,), 1.0, jnp.float32) * val
    for j in range(CHUNK // 16):
        ones_v[pl.ds(j * 16, 16)] = jnp.full((16,), 1.0, jnp.float32)
    pltpu.sync_copy(init_v, deg_sp.at[pl.ds(s * rows_per_tile, rows_per_tile)])
    pltpu.sync_copy(dst_hbm.at[w], dst_v)
    plsc.subcore_barrier()

    def body(j, carry):
        pltpu.sync_copy(ones_v, deg_sp.at[dst_v.at[j]], add=True)
        return carry

    lax.fori_loop(0, cpt, body, 0)
    plsc.subcore_barrier()
    pltpu.sync_copy(deg_sp.at[pl.ds(s * rows_per_tile, rows_per_tile)],
                    degp_hbm.at[c, pl.ds(s * rows_per_tile, rows_per_tile)])


def _y_body(x_ref, w1_ref, d0_ref, d1_ref, y_ref, dinv_ref):
    deg = d0_ref[...] + d1_ref[...]
    dinv = lax.rsqrt(deg)
    xw = jnp.dot(x_ref[...], w1_ref[...], preferred_element_type=jnp.float32)
    y_ref[...] = xw * dinv
    dinv_ref[...] = dinv


def _edge_body(n_pad, rows_per_tile, cpt, src_flat_hbm, dst_hbm, y_hbm,
               zeros_hbm, acc_hbm, src_v, dst_v, buf0, buf1, acc_sp,
               gsem0, gsem1, ssem0, ssem1):
    c = lax.axis_index("c")
    s = lax.axis_index("s")
    w = _wid(c, s)
    pltpu.sync_copy(zeros_hbm, acc_sp.at[pl.ds(s * rows_per_tile, rows_per_tile)])
    pltpu.sync_copy(src_flat_hbm.at[pl.ds(w * cpt * CHUNK, cpt * CHUNK)], src_v)
    pltpu.sync_copy(dst_hbm.at[w], dst_v)
    plsc.subcore_barrier()

    def g_start(j, buf, sem):
        pltpu.async_copy(y_hbm.at[src_v.at[pl.ds(j * CHUNK, CHUNK)]], buf, sem)

    def g_wait(j, buf, sem):
        pltpu.make_async_copy(y_hbm.at[src_v.at[pl.ds(j * CHUNK, CHUNK)]], buf,
                              sem).wait()

    def s_start(j, buf, sem):
        pltpu.async_copy(buf, acc_sp.at[dst_v.at[j]], sem, add=True)

    def s_wait(j, buf, sem):
        pltpu.make_async_copy(buf, acc_sp.at[dst_v.at[j]], sem).wait()

    g_start(0, buf0, gsem0)

    def body(i, carry):
        j0 = 2 * i
        j1 = j0 + 1

        @pl.when(i > 0)
        def _free1():
            s_wait(j0 - 1, buf1, ssem1)

        g_start(j1, buf1, gsem1)
        g_wait(j0, buf0, gsem0)
        s_start(j0, buf0, ssem0)
        s_wait(j0, buf0, ssem0)

        @pl.when(j1 + 1 < cpt)
        def _next0():
            g_start(j1 + 1, buf0, gsem0)

        g_wait(j1, buf1, gsem1)
        s_start(j1, buf1, ssem1)
        return carry

    lax.fori_loop(0, cpt // 2, body, 0)
    s_wait(cpt - 1, buf1, ssem1)
    plsc.subcore_barrier()
    pltpu.sync_copy(acc_sp.at[pl.ds(s * rows_per_tile, rows_per_tile)],
                    acc_hbm.at[c, pl.ds(s * rows_per_tile, rows_per_tile)])


def _final_body(g_segs, a0_ref, a1_ref, y_ref, dinv_ref, batch_ref, b1_ref,
                w2_ref, b2_ref, out_ref, pooled_ref):
    i = pl.program_id(0)

    @pl.when(i == 0)
    def _init():
        pooled_ref[...] = jnp.zeros_like(pooled_ref)

    h = jnp.maximum(
        dinv_ref[...] * (a0_ref[...] + a1_ref[...] + y_ref[...]) + b1_ref[...],
        0.0)
    bids = batch_ref[...]

    def seg(g, carry):
        contrib = jnp.max(jnp.where(bids == g, h, 0.0), axis=0, keepdims=True)
        cur = pooled_ref[pl.ds(g, 1), :]
        pooled_ref[pl.ds(g, 1), :] = jnp.maximum(cur, contrib)
        return carry

    g_lo = batch_ref[0, 0]
    g_hi = batch_ref[batch_ref.shape[0] - 1, 0]
    lax.fori_loop(g_lo, g_hi + 1, seg, 0)

    @pl.when(i == pl.num_programs(0) - 1)
    def _head():
        p = pooled_ref[...]
        out_ref[...] = jnp.maximum(
            jnp.dot(p, w2_ref[...], preferred_element_type=jnp.float32)
            + b2_ref[...], 0.0)


def kernel(x, edge_index, batch, W1, b1, W2, b2):
    n, f = x.shape
    h_dim = W1.shape[1]
    o_dim = W2.shape[1]
    g_segs = 64
    e = edge_index.shape[1]

    cpt = -(-e // (NW * CHUNK))
    cpt = -(-cpt // 2) * 2
    e_pad = NW * cpt * CHUNK
    n_pad = -(-n // (NS * 16)) * (NS * 16)
    rows_per_tile = n_pad // NS
    dummy = n_pad - 1

    src = edge_index[0]
    dst = edge_index[1]
    pad = e_pad - e
    src_p = jnp.concatenate([src, jnp.zeros((pad,), jnp.int32)])
    dst_p = jnp.concatenate(
        [dst, jnp.full((pad,), dummy, jnp.int32)]).reshape(NW, cpt, CHUNK)

    mesh = plsc.VectorSubcoreMesh(core_axis_name="c", subcore_axis_name="s")

    deg_kernel = pl.kernel(
        functools.partial(_deg_body, n_pad, rows_per_tile, cpt),
        out_type=jax.ShapeDtypeStruct((NC, n_pad), jnp.float32),
        mesh=mesh,
        scratch_types=[
            pltpu.VMEM((cpt, CHUNK), jnp.int32),
            pltpu.VMEM((CHUNK,), jnp.float32),
            pltpu.VMEM((rows_per_tile,), jnp.float32),
            pltpu.VMEM_SHARED((n_pad,), jnp.float32),
            pltpu.SemaphoreType.DMA,
        ],
    )
    degp = deg_kernel(dst_p)
    d0 = degp[0, :n].reshape(n, 1)
    d1 = degp[1, :n].reshape(n, 1)

    y, dinv = pl.pallas_call(
        _y_body,
        grid=(n // RBLK,),
        in_specs=[
            pl.BlockSpec((RBLK, f), lambda i: (i, 0)),
            pl.BlockSpec((f, h_dim), lambda i: (0, 0)),
            pl.BlockSpec((RBLK, 1), lambda i: (i, 0)),
            pl.BlockSpec((RBLK, 1), lambda i: (i, 0)),
        ],
        out_specs=[
            pl.BlockSpec((RBLK, h_dim), lambda i: (i, 0)),
            pl.BlockSpec((RBLK, 1), lambda i: (i, 0)),
        ],
        out_shape=[
            jax.ShapeDtypeStruct((n, h_dim), jnp.float32),
            jax.ShapeDtypeStruct((n, 1), jnp.float32),
        ],
    )(x, W1, d0, d1)

    zeros_rows = jnp.zeros((rows_per_tile, h_dim), jnp.float32)
    edge_kernel = pl.kernel(
        functools.partial(_edge_body, n_pad, rows_per_tile, cpt),
        out_type=jax.ShapeDtypeStruct((NC, n_pad, h_dim), jnp.float32),
        mesh=mesh,
        scratch_types=[
            pltpu.VMEM((cpt * CHUNK,), jnp.int32),
            pltpu.VMEM((cpt, CHUNK), jnp.int32),
            pltpu.VMEM((CHUNK, h_dim), jnp.float32),
            pltpu.VMEM((CHUNK, h_dim), jnp.float32),
            pltpu.VMEM_SHARED((n_pad, h_dim), jnp.float32),
            pltpu.SemaphoreType.DMA,
            pltpu.SemaphoreType.DMA,
            pltpu.SemaphoreType.DMA,
            pltpu.SemaphoreType.DMA,
        ],
    )
    acc = edge_kernel(src_p, dst_p, y, zeros_rows)
    a0 = acc[0, :n, :]
    a1 = acc[1, :n, :]

    out = pl.pallas_call(
        functools.partial(_final_body, g_segs),
        grid=(n // RBLK,),
        in_specs=[
            pl.BlockSpec((RBLK, h_dim), lambda i: (i, 0)),
            pl.BlockSpec((RBLK, h_dim), lambda i: (i, 0)),
            pl.BlockSpec((RBLK, h_dim), lambda i: (i, 0)),
            pl.BlockSpec((RBLK, 1), lambda i: (i, 0)),
            pl.BlockSpec((RBLK, 1), lambda i: (i, 0)),
            pl.BlockSpec((1, h_dim), lambda i: (0, 0)),
            pl.BlockSpec((h_dim, o_dim), lambda i: (0, 0)),
            pl.BlockSpec((1, o_dim), lambda i: (0, 0)),
        ],
        out_specs=pl.BlockSpec((g_segs, o_dim), lambda i: (0, 0)),
        out_shape=jax.ShapeDtypeStruct((g_segs, o_dim), jnp.float32),
        scratch_shapes=[pltpu.VMEM((g_segs, h_dim), jnp.float32)],
    )(a0, a1, y, dinv, batch.reshape(n, 1), b1.reshape(1, h_dim), W2,
      b2.reshape(1, o_dim))
    return out
